# Initial kernel scaffold; baseline (speedup 1.0000x reference)
import jax
import jax.numpy as jnp
from jax.experimental import pallas as pl


def kernel(x, edge_index, batch, W1_0, b1_0, W2_0, b2_0, eps_0, W1_1, b1_1, W2_1, b2_1, eps_1, W1_2, b1_2, W2_2, b2_2, eps_2):
    raise NotImplementedError("write your pallas kernel here")


# speedup vs baseline: 5.7393x; 5.7393x over previous
import functools

import jax
import jax.numpy as jnp
from jax import lax
from jax.experimental import pallas as pl
from jax.experimental.pallas import tpu as pltpu
from jax.experimental.pallas import tpu_sc as plsc

NC = 2
NS = 16
NW = NC * NS

G = 128


def _sc_aggregate(h, src, dst, zeros_nd):
  N, D = h.shape
  NP = zeros_nd.shape[0]
  E = src.shape[0]
  EPT = E // NW
  CH = 128
  NFULL = EPT // CH
  REM = EPT - NFULL * CH
  RPT = NP // NS

  mesh = plsc.VectorSubcoreMesh(core_axis_name="c", subcore_axis_name="s")

  scratch = [
      pltpu.VMEM((CH,), jnp.int32),
      pltpu.VMEM((CH,), jnp.int32),
      pltpu.VMEM((CH, D), jnp.float32),
      pltpu.VMEM_SHARED((NP, D), jnp.float32),
      pltpu.SemaphoreType.DMA,
  ]
  if REM:
    scratch += [
        pltpu.VMEM((REM,), jnp.int32),
        pltpu.VMEM((REM,), jnp.int32),
        pltpu.VMEM((REM, D), jnp.float32),
    ]

  @functools.partial(
      pl.kernel,
      mesh=mesh,
      out_type=jax.ShapeDtypeStruct((NC, NP, D), jnp.float32),
      scratch_types=scratch,
  )
  def agg_kernel(h_hbm, src_hbm, dst_hbm, z_hbm, out_hbm,
                 sidx, didx, rows, acc_sh, sem, *rem_scratch):
    cid = lax.axis_index("c")
    sid = lax.axis_index("s")
    wid = sid * NC + cid

    pltpu.sync_copy(z_hbm.at[pl.ds(sid * RPT, RPT)],
                    acc_sh.at[pl.ds(sid * RPT, RPT)])
    plsc.subcore_barrier()

    base = wid * EPT

    def body(g, carry):
      off = base + g * CH
      pltpu.sync_copy(src_hbm.at[pl.ds(off, CH)], sidx)
      pltpu.async_copy(h_hbm.at[sidx], rows, sem).wait()
      pltpu.sync_copy(dst_hbm.at[pl.ds(off, CH)], didx)
      pltpu.sync_copy(rows, acc_sh.at[didx], add=True)
      return carry

    lax.fori_loop(0, NFULL, body, 0)

    if REM:
      sidx2, didx2, rows2 = rem_scratch
      off = base + NFULL * CH
      pltpu.sync_copy(src_hbm.at[pl.ds(off, REM)], sidx2)
      pltpu.async_copy(h_hbm.at[sidx2], rows2, sem).wait()
      pltpu.sync_copy(dst_hbm.at[pl.ds(off, REM)], didx2)
      pltpu.sync_copy(rows2, acc_sh.at[didx2], add=True)

    plsc.subcore_barrier()
    pltpu.sync_copy(acc_sh.at[pl.ds(sid * RPT, RPT)],
                    out_hbm.at[cid, pl.ds(sid * RPT, RPT)])

  return agg_kernel(h, src, dst, zeros_nd)


def _tc_layer(h, parts, batch3, W1, b1r, W2, b2r, eps11, B):
  N, D = h.shape
  H = W1.shape[1]
  NB = N // B

  def body(eps_ref, x_ref, a0_ref, a1_ref, b_ref, w1_ref, b1_ref,
           w2_ref, b2_ref, nodes_ref, pool_ref, cnt_ref):
    i = pl.program_id(0)
    e = eps_ref[0, 0]
    hin = (1.0 + e) * x_ref[...] + a0_ref[0] + a1_ref[0]
    t = jnp.dot(hin, w1_ref[...], preferred_element_type=jnp.float32)
    t = jnp.maximum(t + b1_ref[...], 0.0)
    out = jnp.dot(t, w2_ref[...], preferred_element_type=jnp.float32)
    out = jnp.maximum(out + b2_ref[...], 0.0)
    nodes_ref[...] = out

    bids = b_ref[0, 0, :]
    gids = lax.broadcasted_iota(jnp.int32, (G, B), 0)
    mask = (bids[None, :] == gids).astype(jnp.float32)

    @pl.when(i == 0)
    def _():
      pool_ref[...] = jnp.zeros_like(pool_ref)
      cnt_ref[...] = jnp.zeros_like(cnt_ref)

    pool_ref[...] += jnp.dot(mask, out, preferred_element_type=jnp.float32)
    cnt_ref[...] += jnp.sum(mask, axis=1, keepdims=True)

    @pl.when(i == NB - 1)
    def _():
      pool_ref[...] = pool_ref[...] / jnp.maximum(cnt_ref[...], 1.0)

  nodes, pool = pl.pallas_call(
      body,
      grid=(NB,),
      in_specs=[
          pl.BlockSpec(memory_space=pltpu.SMEM),
          pl.BlockSpec((B, D), lambda i: (i, 0)),
          pl.BlockSpec((1, B, D), lambda i: (0, i, 0)),
          pl.BlockSpec((1, B, D), lambda i: (1, i, 0)),
          pl.BlockSpec((1, 1, B), lambda i: (i, 0, 0)),
          pl.BlockSpec((D, H), lambda i: (0, 0)),
          pl.BlockSpec((1, H), lambda i: (0, 0)),
          pl.BlockSpec((H, H), lambda i: (0, 0)),
          pl.BlockSpec((1, H), lambda i: (0, 0)),
      ],
      out_specs=[
          pl.BlockSpec((B, H), lambda i: (i, 0)),
          pl.BlockSpec((G, H), lambda i: (0, 0)),
      ],
      out_shape=[
          jax.ShapeDtypeStruct((N, H), jnp.float32),
          jax.ShapeDtypeStruct((G, H), jnp.float32),
      ],
      scratch_shapes=[pltpu.VMEM((G, 1), jnp.float32)],
  )(eps11, h, parts, parts, batch3, W1, b1r, W2, b2r)
  return nodes, pool


def kernel(x, edge_index, batch,
           W1_0, b1_0, W2_0, b2_0, eps_0,
           W1_1, b1_1, W2_1, b2_1, eps_1,
           W1_2, b1_2, W2_2, b2_2, eps_2):
  N, D = x.shape
  src = edge_index[0].astype(jnp.int32)
  dst = edge_index[1].astype(jnp.int32)
  B = 1000
  batch3 = batch.astype(jnp.int32).reshape(N // B, 1, B)
  NP = ((N + 8 * NS - 1) // (8 * NS)) * (8 * NS)
  zeros_nd = jnp.zeros((NP, D), jnp.float32)

  params = [
      (W1_0, b1_0.reshape(1, -1), W2_0, b2_0.reshape(1, -1),
       eps_0.reshape(1, 1)),
      (W1_1, b1_1.reshape(1, -1), W2_1, b2_1.reshape(1, -1),
       eps_1.reshape(1, 1)),
      (W1_2, b1_2.reshape(1, -1), W2_2, b2_2.reshape(1, -1),
       eps_2.reshape(1, 1)),
  ]

  h = x
  nodes_list = []
  pool_list = []
  for (W1, b1r, W2, b2r, eps11) in params:
    parts = _sc_aggregate(h, src, dst, zeros_nd)
    h, pool = _tc_layer(h, parts, batch3, W1, b1r, W2, b2r, eps11, B)
    nodes_list.append(h)
    pool_list.append(pool)

  out_pool = jnp.concatenate(pool_list, axis=1)
  out_nodes = jnp.concatenate(nodes_list, axis=1)
  return (out_pool, out_nodes)
